# baseline (speedup 1.0000x reference)
import functools

import jax
import jax.numpy as jnp
from jax import lax
from jax.experimental import pallas as pl
from jax.experimental.pallas import tpu as pltpu
from jax.experimental.pallas import tpu_sc as plsc

_HARD = 0.9
_SOFT = 0.02
_K = 5


def _sc_tail(logits, labels, *, B, VTC0, VSC, CH):
    L = 16
    GROUP = 16
    NCHUNK = VSC // CH
    UNROLL = 8
    NIT = CH // (L * UNROLL)
    info = plsc.get_sparse_core_info()
    NC, NS = info.num_cores, info.num_subcores
    NW = NC * NS
    RPW = B // NW
    NGRP = RPW // GROUP
    NQ = NGRP * NCHUNK

    mesh = plsc.VectorSubcoreMesh(core_axis_name="c", subcore_axis_name="s")

    @functools.partial(
        pl.kernel, mesh=mesh,
        compiler_params=pltpu.CompilerParams(use_tc_tiling_on_sc=True),
        out_type=(jax.ShapeDtypeStruct((B, L), jnp.float32),
                  jax.ShapeDtypeStruct((B, L), jnp.float32),
                  jax.ShapeDtypeStruct((B, L), jnp.float32)),
        scratch_types=[
            pltpu.VMEM((GROUP, CH), jnp.float32),
            pltpu.VMEM((GROUP, CH), jnp.float32),
            pltpu.VMEM((RPW, L), jnp.float32),
            pltpu.VMEM((RPW, L), jnp.float32),
            pltpu.VMEM((RPW, L), jnp.float32),
            pltpu.VMEM((RPW + L,), jnp.int32),
            pltpu.SemaphoreType.DMA,
            pltpu.SemaphoreType.DMA,
        ],
    )
    def k(logits_hbm, labels_hbm, m_hbm, s_hbm, lab_hbm, buf0, buf1,
          stage_m, stage_s, stage_l, lbuf, sem0, sem1):
        wid = lax.axis_index("s") * NC + lax.axis_index("c")
        row0 = wid * RPW
        lane = lax.broadcasted_iota(jnp.int32, (L,), 0)
        bufs = (buf0, buf1)
        sems = (sem0, sem1)

        def src(q):
            return logits_hbm.at[pl.ds(row0 + (q // NCHUNK) * GROUP, GROUP),
                                 pl.ds((q % NCHUNK) * CH, CH)]

        pltpu.sync_copy(labels_hbm.at[pl.ds(row0, RPW)],
                        lbuf.at[pl.ds(0, RPW)])
        pltpu.async_copy(src(0), buf0, sem0)
        pltpu.async_copy(src(1), buf1, sem1)

        def init_body(r, carry):
            stage_m[r, :] = jnp.full((L,), -jnp.inf, jnp.float32)
            stage_s[r, :] = jnp.zeros((L,), jnp.float32)
            stage_l[r, :] = jnp.zeros((L,), jnp.float32)
            return carry

        lax.fori_loop(0, RPW, init_body, 0)

        def do_chunk(buf, q):
            g = q // NCHUNK
            c = q % NCHUNK
            col0 = VTC0 + c * CH

            def row_body(rr, carry, buf=buf):
                r = g * GROUP + rr
                m16 = stage_m[r, :]
                s16 = stage_s[r, :]
                lab16 = stage_l[r, :]

                def max_body(i, mloc, buf=buf, rr=rr):
                    for u in range(UNROLL):
                        mloc = jnp.maximum(
                            mloc, buf[rr, pl.ds(i * (UNROLL * L) + u * L, L)])
                    return mloc
                mloc = lax.fori_loop(0, NIT, max_body,
                                     jnp.full((L,), -jnp.inf, jnp.float32))
                m16n = jnp.maximum(m16, mloc)
                s16 = s16 * jnp.exp(m16 - m16n)

                def exp_body(i, s16, buf=buf, rr=rr, m16n=m16n):
                    for u in range(UNROLL):
                        s16 = s16 + jnp.exp(
                            buf[rr, pl.ds(i * (UNROLL * L) + u * L, L)] - m16n)
                    return s16
                s16 = lax.fori_loop(0, NIT, exp_body, s16)

                lab = lbuf[pl.ds(r, L)][0]
                off = lab - col0
                sel = (off >= 0) & (off < CH)
                a = jnp.minimum(jnp.maximum(off, 0) // L * L, CH - L)
                chunk = buf[rr, pl.ds(a, L)]
                target = jnp.where(sel, off - a, -1)
                lab16 = jnp.where(lane == target, chunk, lab16)

                stage_m[r, :] = m16n
                stage_s[r, :] = s16
                stage_l[r, :] = lab16
                return carry

            lax.fori_loop(0, GROUP, row_body, 0)

        def pair_body(p, carry):
            q0 = p * 2
            pltpu.make_async_copy(src(q0), buf0, sem0).wait()
            do_chunk(buf0, q0)

            @pl.when(q0 + 2 < NQ)
            def _():
                pltpu.async_copy(src(q0 + 2), buf0, sem0)

            pltpu.make_async_copy(src(q0 + 1), buf1, sem1).wait()
            do_chunk(buf1, q0 + 1)

            @pl.when(q0 + 3 < NQ)
            def _():
                pltpu.async_copy(src(q0 + 3), buf1, sem1)
            return carry

        lax.fori_loop(0, NQ // 2, pair_body, 0)

        pltpu.sync_copy(stage_m, m_hbm.at[pl.ds(row0, RPW)])
        pltpu.sync_copy(stage_s, s_hbm.at[pl.ds(row0, RPW)])
        pltpu.sync_copy(stage_l, lab_hbm.at[pl.ds(row0, RPW)])

    return k(logits, labels)


def _tc_main_body(labels_ref, x_ref, out_ref, m_ref, s_ref, lab_ref, t5_ref,
                  *, V, RAG, NV):
    j = pl.program_id(1)
    Rb = x_ref.shape[0]

    @pl.when(j == 0)
    def _init():
        m_ref[...] = jnp.full_like(m_ref, -jnp.inf)
        s_ref[...] = jnp.zeros_like(s_ref)
        lab_ref[...] = jnp.zeros_like(lab_ref)
        t5_ref[...] = jnp.full_like(t5_ref, -jnp.inf)

    x = x_ref[...]
    Vb = x.shape[1]
    jm = jnp.where(j == NV - 1, RAG, j)
    cols = lax.broadcasted_iota(jnp.int32, x.shape, 1) + jm * Vb
    x = lax.cond(j == NV - 1,
                 lambda v: jnp.where(cols < V, v, -jnp.inf),
                 lambda v: v, x)

    bmax = jnp.max(x, axis=1, keepdims=True)
    m_old = m_ref[...]
    m_new = jnp.maximum(m_old, bmax)
    e = jnp.exp(x - m_new)
    s_ref[...] = s_ref[...] * jnp.exp(m_old - m_new) + jnp.sum(
        e, axis=1, keepdims=True)
    m_ref[...] = m_new

    hit = cols == labels_ref[...]
    lab_ref[...] = lab_ref[...] + jnp.sum(jnp.where(hit, x, 0.0), axis=1,
                                          keepdims=True)

    y = x
    w = Vb
    while w > 128:
        w //= 2
        y = jnp.maximum(y[:, :w], y[:, w:2 * w])
    vals = []
    for _ in range(_K):
        v = jnp.max(y, axis=1, keepdims=True)
        vals.append(v)
        y = jnp.where(y >= v, -jnp.inf, y)
    z = jnp.concatenate(vals + [t5_ref[...]], axis=1)
    vals2 = []
    for _ in range(_K):
        v = jnp.max(z, axis=1, keepdims=True)
        vals2.append(v)
        z = jnp.where(z >= v, -jnp.inf, z)
    t5_new = jnp.concatenate(
        vals2 + [jnp.full((Rb, 8 - _K), -jnp.inf, x.dtype)], axis=1)
    t5_ref[...] = t5_new

    @pl.when(j == NV - 1)
    def _finish():
        out_ref[...] = jnp.concatenate(
            [m_new, s_ref[...], t5_new[:, :_K], lab_ref[...]], axis=1)


def _tc_main(logits, labels2, *, VTC0, Rb, Vb):
    B, V = logits.shape
    NFULL = VTC0 // Vb
    RAG = (V - 1) // Vb
    NV = NFULL + 1
    body = functools.partial(_tc_main_body, V=V, RAG=RAG, NV=NV)
    return pl.pallas_call(
        body,
        grid=(B // Rb, NV),
        in_specs=[pl.BlockSpec((Rb, 1), lambda i, j: (i, 0)),
                  pl.BlockSpec((Rb, Vb),
                               lambda i, j: (i, jnp.where(j == NFULL, RAG, j)))],
        out_specs=pl.BlockSpec((Rb, 8), lambda i, j: (i, 0)),
        out_shape=jax.ShapeDtypeStruct((B, 8), jnp.float32),
        scratch_shapes=[pltpu.VMEM((Rb, 1), jnp.float32),
                        pltpu.VMEM((Rb, 1), jnp.float32),
                        pltpu.VMEM((Rb, 1), jnp.float32),
                        pltpu.VMEM((Rb, 8), jnp.float32)],
        compiler_params=pltpu.CompilerParams(
            dimension_semantics=("parallel", "arbitrary")),
    )(labels2, logits)


def _merge_body(tc_ref, m16_ref, s16_ref, lab16_ref, out_ref):
    tc = tc_ref[...]
    m16 = m16_ref[...]
    s16 = s16_ref[...]
    m_tc, s_tc = tc[:, 0:1], tc[:, 1:2]
    labv = tc[:, 7:8] + jnp.sum(lab16_ref[...], axis=1, keepdims=True)
    m_sc = jnp.max(m16, axis=1, keepdims=True)
    s_sc = jnp.sum(s16 * jnp.exp(m16 - m_sc), axis=1, keepdims=True)
    m = jnp.maximum(m_tc, m_sc)
    s = s_tc * jnp.exp(m_tc - m) + s_sc * jnp.exp(m_sc - m)
    lse = m + jnp.log(s)
    z = jnp.concatenate([tc[:, 2:7], m16], axis=1)
    sum5 = jnp.zeros_like(m)
    for _ in range(_K):
        v = jnp.max(z, axis=1, keepdims=True)
        sum5 = sum5 + v
        z = jnp.where(z >= v, -jnp.inf, z)
    out_ref[...] = lse - _HARD * labv - _SOFT * sum5


def _merge(tc_out, m_sc, s_sc, lab_sc):
    B = tc_out.shape[0]
    Rb = 256
    out = pl.pallas_call(
        _merge_body,
        grid=(B // Rb,),
        in_specs=[pl.BlockSpec((Rb, 8), lambda i: (i, 0)),
                  pl.BlockSpec((Rb, 16), lambda i: (i, 0)),
                  pl.BlockSpec((Rb, 16), lambda i: (i, 0)),
                  pl.BlockSpec((Rb, 16), lambda i: (i, 0))],
        out_specs=pl.BlockSpec((Rb, 1), lambda i: (i, 0)),
        out_shape=jax.ShapeDtypeStruct((B, 1), jnp.float32),
    )(tc_out, m_sc, s_sc, lab_sc)
    return out.reshape(B)


def kernel(logits, labels):
    B, V = logits.shape
    Vb = 8192
    VTC0 = 9 * Vb
    VSC = 24576
    CH = 2048
    Rb = 128
    labels_i = labels.astype(jnp.int32)
    sc_slice = jax.lax.slice(logits, (0, VTC0), (B, VTC0 + VSC))
    m_sc, s_sc, lab_sc = _sc_tail(sc_slice, labels_i, B=B, VTC0=VTC0, VSC=VSC,
                                  CH=CH)
    tc_out = _tc_main(logits, labels_i.reshape(B, 1), VTC0=VTC0, Rb=Rb, Vb=Vb)
    return _merge(tc_out, m_sc, s_sc, lab_sc)


# speedup vs baseline: 1.1988x; 1.1988x over previous
import functools

import jax
import jax.numpy as jnp
from jax.experimental import pallas as pl
from jax.experimental.pallas import tpu as pltpu

_HARD = 0.9
_SOFT = 0.02
_K = 5


def _loss_body(labels_ref, logits_ref, out_ref, m_ref, s_ref, lab_ref, t5_ref,
               *, V, Vb, NV):
    j = pl.program_id(1)

    @pl.when(j == 0)
    def _init():
        m_ref[...] = jnp.full_like(m_ref, -jnp.inf)
        s_ref[...] = jnp.zeros_like(s_ref)
        lab_ref[...] = jnp.zeros_like(lab_ref)
        t5_ref[...] = jnp.full_like(t5_ref, -jnp.inf)

    x = logits_ref[...]
    Rb = x.shape[0]
    cols = jax.lax.broadcasted_iota(jnp.int32, x.shape, 1) + j * Vb
    x = jnp.where(cols < V, x, -jnp.inf)

    bmax = jnp.max(x, axis=1, keepdims=True)
    m_old = m_ref[...]
    m_new = jnp.maximum(m_old, bmax)
    e = jnp.exp(x - m_new)
    s_ref[...] = s_ref[...] * jnp.exp(m_old - m_new) + jnp.sum(e, axis=1, keepdims=True)
    m_ref[...] = m_new

    lab = labels_ref[...]
    hit = cols == lab
    lab_ref[...] = lab_ref[...] + jnp.sum(jnp.where(hit, x, 0.0), axis=1,
                                          keepdims=True)

    y = x
    w = Vb
    while w > 128:
        w //= 2
        y = jnp.maximum(y[:, :w], y[:, w:2 * w])
    vals = []
    for _ in range(_K):
        v = jnp.max(y, axis=1, keepdims=True)
        vals.append(v)
        y = jnp.where(y >= v, -jnp.inf, y)
    z = jnp.concatenate(vals + [t5_ref[...]], axis=1)
    vals2 = []
    for _ in range(_K):
        v = jnp.max(z, axis=1, keepdims=True)
        vals2.append(v)
        z = jnp.where(z >= v, -jnp.inf, z)
    t5_new = jnp.concatenate(
        vals2 + [jnp.full((Rb, 8 - _K), -jnp.inf, x.dtype)], axis=1)
    t5_ref[...] = t5_new

    @pl.when(j == NV - 1)
    def _finish():
        lse = m_ref[...] + jnp.log(s_ref[...])
        sum5 = jnp.sum(t5_new[:, :_K], axis=1, keepdims=True)
        out_ref[...] = lse - _HARD * lab_ref[...] - _SOFT * sum5


def kernel(logits, labels):
    B, V = logits.shape
    Rb = 128 if B % 128 == 0 else 8
    Vb = 8192 if V >= 8192 else 128
    NV = (V + Vb - 1) // Vb

    labels2 = labels.reshape(B, 1).astype(jnp.int32)
    body = functools.partial(_loss_body, V=V, Vb=Vb, NV=NV)
    out = pl.pallas_call(
        body,
        grid=(B // Rb, NV),
        in_specs=[
            pl.BlockSpec((Rb, 1), lambda i, j: (i, 0)),
            pl.BlockSpec((Rb, Vb), lambda i, j: (i, j)),
        ],
        out_specs=pl.BlockSpec((Rb, 1), lambda i, j: (i, 0)),
        out_shape=jax.ShapeDtypeStruct((B, 1), logits.dtype),
        scratch_shapes=[
            pltpu.VMEM((Rb, 1), jnp.float32),
            pltpu.VMEM((Rb, 1), jnp.float32),
            pltpu.VMEM((Rb, 1), jnp.float32),
            pltpu.VMEM((Rb, 8), jnp.float32),
        ],
        compiler_params=pltpu.CompilerParams(
            dimension_semantics=("parallel", "arbitrary")),
    )(labels2, logits)
    return out.reshape(B)
